# baseline (speedup 1.0000x reference)
import functools

import jax
import jax.numpy as jnp
from jax import lax
from jax.experimental import pallas as pl
from jax.experimental.pallas import tpu as pltpu, tpu_sc as plsc

NC = 2
NS = 16
NW = NC * NS
L = 16
B = 16384
D = 64
ROWS_PER_TILE = B // NW


def _repack_body(nblk, nsrc, src_hbm, dst_hbm,
                 vin0, vin1, vout0, vout1, sem_i, sem_o):
    c = lax.axis_index("c")
    s = lax.axis_index("s")
    ot = c * NS + s
    per = (nblk + NW - 1) // NW
    lo = ot * per
    hi = jnp.minimum(lo + per, nblk)
    n_mine = jnp.maximum(hi - lo, 0)
    vins = (vin0, vin1)
    vouts = (vout0, vout1)
    rows = jnp.arange(L, dtype=jnp.int32)
    def fetch(blk, buf):
        pltpu.async_copy(src_hbm.at[:, pl.ds(blk * 128, 128)], buf, sem_i)

    def fetch_wait(blk, buf):
        pltpu.make_async_copy(
            src_hbm.at[:, pl.ds(0, 128)], buf, sem_i).wait()

    def transpose(vin, vout):
        def pbody(p, _):
            for h in range(2):
                col = jnp.full((L,), 2 * p + h, jnp.int32)
                for q in range(D // L):
                    v = plsc.load_gather(vin, [rows + q * L, col])
                    vout[p, pl.ds(h * D + q * L, L)] = v
            return 0

        lax.fori_loop(0, D, pbody, 0, unroll=8)

    def store(blk, buf):
        pltpu.async_copy(buf, dst_hbm.at[pl.ds(blk * D, D), :], sem_o)

    def store_wait(buf):
        pltpu.make_async_copy(dst_hbm.at[pl.ds(0, D), :], buf, sem_o).wait()

    @pl.when(lo < nblk)
    def _():
        fetch(lo, vins[0])

    def body(i, _):
        for par in range(2):
            j = i * 2 + par
            blk = lo + j

            @pl.when(blk < hi)
            def _():
                fetch_wait(blk, vins[par])

                @pl.when(blk + 1 < hi)
                def _():
                    fetch(blk + 1, vins[1 - par])

                @pl.when(j >= 2)
                def _():
                    store_wait(vouts[par])

                transpose(vins[par], vouts[par])
                store(blk, vouts[par])
        return 0

    lax.fori_loop(0, (per + 1) // 2, body, 0)

    @pl.when(n_mine >= 1)
    def _():
        store_wait(vouts[0])

    @pl.when(n_mine >= 2)
    def _():
        store_wait(vouts[1])


def _dot_body(uidx_hbm, midx_hbm, ut2_hbm, mt2_hbm, part_hbm,
              idx_u_v, idx_m_v, pidx_v, ubuf, mbuf, stage_v, sem_u, sem_m):
    c = lax.axis_index("c")
    s = lax.axis_index("s")
    ot = c * NS + s

    for j in range(ROWS_PER_TILE // 128):
        pltpu.sync_copy(uidx_hbm.at[ot * 4 + j], idx_u_v.at[pl.ds(j * 128, 128)])
        pltpu.sync_copy(midx_hbm.at[ot * 4 + j], idx_m_v.at[pl.ds(j * 128, 128)])
    for k in range(ROWS_PER_TILE // L):
        sl = pl.ds(k * L, L)
        pidx_v[sl] = idx_u_v[sl] >> 1
        pidx_v[pl.ds(ROWS_PER_TILE + k * L, L)] = idx_m_v[sl] >> 1

    def issue(j):
        bb = j % 2
        du = pltpu.async_copy(
            ut2_hbm.at[pidx_v.at[pl.ds(j * 128, 128)]], ubuf.at[bb], sem_u)
        dm = pltpu.async_copy(
            mt2_hbm.at[pidx_v.at[pl.ds(ROWS_PER_TILE + j * 128, 128)]],
            mbuf.at[bb], sem_m)
        return du, dm

    descs = [issue(0), issue(1)]
    acc = jnp.zeros((L,), jnp.float32)
    for j in range(ROWS_PER_TILE // 128):
        bb = j % 2
        du, dm = descs[j]
        du.wait()
        dm.wait()

        def gbody(k, acc, j=j, bb=bb):
            pu = idx_u_v[pl.ds(j * 128 + k * L, L)] & 1
            pm = idx_m_v[pl.ds(j * 128 + k * L, L)] & 1
            for t in range(L):
                r = k * L + t
                hu = pu[t] * D
                hm = pm[t] * D
                for q in range(D // L):
                    acc = acc + (ubuf[bb, r, pl.ds(hu + q * L, L)]
                                 * mbuf[bb, r, pl.ds(hm + q * L, L)])
            return acc

        acc = lax.fori_loop(0, 128 // L, gbody, acc)
        if j + 2 < ROWS_PER_TILE // 128:
            descs.append(issue(j + 2))

    stage_v[...] = acc
    pltpu.sync_copy(stage_v, part_hbm.at[ot])


def _out_body(uidx_hbm, midx_hbm, ub_hbm, mb_hbm, gb_hbm, part_hbm, out_hbm,
              idx_u_v, idx_m_v, ubf, mbf, out_v, part_v, gb_v,
              sem_u, sem_m):
    c = lax.axis_index("c")
    s = lax.axis_index("s")
    ot = c * NS + s

    pltpu.sync_copy(part_hbm, part_v)
    pltpu.sync_copy(gb_hbm, gb_v)
    for j in range(ROWS_PER_TILE // 128):
        pltpu.sync_copy(uidx_hbm.at[ot * 4 + j], idx_u_v.at[pl.ds(j * 128, 128)])
        pltpu.sync_copy(midx_hbm.at[ot * 4 + j], idx_m_v.at[pl.ds(j * 128, 128)])
    descs = []
    for j in range(ROWS_PER_TILE // 128):
        sl = pl.ds(j * 128, 128)
        descs.append(pltpu.async_copy(
            ub_hbm.at[idx_u_v.at[sl]], ubf.at[sl], sem_u))
        descs.append(pltpu.async_copy(
            mb_hbm.at[idx_m_v.at[sl]], mbf.at[sl], sem_m))

    tot = part_v[0]
    for q in range(1, NW):
        tot = tot + part_v[q]
    dot = tot[0]
    for q in range(1, L):
        dot = dot + tot[q]
    base = dot + gb_v[...]

    for d in descs:
        d.wait()
    for k in range(ROWS_PER_TILE // L):
        sl = pl.ds(k * L, L)
        out_v[sl] = ubf[sl] + mbf[sl] + base
    pltpu.sync_copy(out_v, out_hbm.at[pl.ds(ot * ROWS_PER_TILE, ROWS_PER_TILE)])


def _repack(src_t, nblk, nsrc, nrows):
    mesh = plsc.VectorSubcoreMesh(core_axis_name="c", subcore_axis_name="s")
    return functools.partial(
        pl.kernel,
        out_type=jax.ShapeDtypeStruct((nrows, 128), jnp.float32),
        mesh=mesh,
        compiler_params=pltpu.CompilerParams(
            use_tc_tiling_on_sc=True, needs_layout_passes=False),
        scratch_types=[
            pltpu.VMEM((D, 128), jnp.float32),
            pltpu.VMEM((D, 128), jnp.float32),
            pltpu.VMEM((D, 128), jnp.float32),
            pltpu.VMEM((D, 128), jnp.float32),
            pltpu.SemaphoreType.DMA,
            pltpu.SemaphoreType.DMA,
        ],
    )(functools.partial(_repack_body, nblk, nsrc))(src_t)


@jax.jit
def _run(uidx_r, midx_r, user_t, movie_t, ubias, mbias, gb_vec):
    mesh = plsc.VectorSubcoreMesh(core_axis_name="c", subcore_axis_name="s")
    ut2 = _repack(user_t, 782, 100000, 50048)
    mt2 = _repack(movie_t, 7813, 1000000, 500032)

    part = functools.partial(
        pl.kernel,
        out_type=jax.ShapeDtypeStruct((NW, L), jnp.float32),
        mesh=mesh,
        compiler_params=pltpu.CompilerParams(use_tc_tiling_on_sc=True),
        scratch_types=[
            pltpu.VMEM((ROWS_PER_TILE,), jnp.int32),
            pltpu.VMEM((ROWS_PER_TILE,), jnp.int32),
            pltpu.VMEM((2 * ROWS_PER_TILE,), jnp.int32),
            pltpu.VMEM((2, 128, 128), jnp.float32),
            pltpu.VMEM((2, 128, 128), jnp.float32),
            pltpu.VMEM((L,), jnp.float32),
            pltpu.SemaphoreType.DMA,
            pltpu.SemaphoreType.DMA,
        ],
    )(_dot_body)(uidx_r, midx_r, ut2, mt2)

    out = functools.partial(
        pl.kernel,
        out_type=jax.ShapeDtypeStruct((B,), jnp.float32),
        mesh=mesh,
        compiler_params=pltpu.CompilerParams(use_tc_tiling_on_sc=False),
        scratch_types=[
            pltpu.VMEM((ROWS_PER_TILE,), jnp.int32),
            pltpu.VMEM((ROWS_PER_TILE,), jnp.int32),
            pltpu.VMEM((ROWS_PER_TILE,), jnp.float32),
            pltpu.VMEM((ROWS_PER_TILE,), jnp.float32),
            pltpu.VMEM((ROWS_PER_TILE,), jnp.float32),
            pltpu.VMEM((NW, L), jnp.float32),
            pltpu.VMEM((L,), jnp.float32),
            pltpu.SemaphoreType.DMA,
            pltpu.SemaphoreType.DMA,
        ],
    )(_out_body)(uidx_r, midx_r, ubias, mbias, gb_vec, part)
    return out


def kernel(inputs, user_emb, movie_emb, user_bias_table, movie_bias_table,
           global_bias):
    uidx = inputs[:, 0].reshape(B // 128, 128)
    midx = inputs[:, 1].reshape(B // 128, 128)
    gb_vec = jnp.full((L,), global_bias, dtype=jnp.float32)
    return _run(uidx, midx, user_emb.T, movie_emb.T,
                user_bias_table.reshape(-1), movie_bias_table.reshape(-1),
                gb_vec)


# speedup vs baseline: 3.7724x; 3.7724x over previous
import functools

import jax
import jax.numpy as jnp
from jax import lax
from jax.experimental import pallas as pl
from jax.experimental.pallas import tpu as pltpu, tpu_sc as plsc

NC = 2
NS = 16
L = 16
B = 16384
D = 64
ROWS_PER_TILE = B // (NC * NS)
GROUPS = ROWS_PER_TILE // L
NBUF = 2


def _dot_body(uidx_hbm, midx_hbm, ue_hbm, me_hbm, part_hbm,
              idx_u_v, idx_m_v, ubuf, mbuf, stage_v, sem_u, sem_m):
    c = lax.axis_index("c")
    s = lax.axis_index("s")
    ot = c * NS + s

    for j in range(ROWS_PER_TILE // 128):
        pltpu.sync_copy(uidx_hbm.at[ot * 4 + j], idx_u_v.at[pl.ds(j * 128, 128)])
        pltpu.sync_copy(midx_hbm.at[ot * 4 + j], idx_m_v.at[pl.ds(j * 128, 128)])

    def issue(g, b):
        iv_u = idx_u_v[pl.ds(g * L, L)]
        iv_m = idx_m_v[pl.ds(g * L, L)]
        for t in range(L):
            bu = pl.multiple_of((iv_u[t] >> 3) * 8, 8)
            bm = pl.multiple_of((iv_m[t] >> 3) * 8, 8)
            pltpu.async_copy(ue_hbm.at[pl.ds(bu, 8), :], ubuf.at[b, t], sem_u)
            pltpu.async_copy(me_hbm.at[pl.ds(bm, 8), :], mbuf.at[b, t], sem_m)

    def wait_group(b):
        for t in range(L):
            pltpu.make_async_copy(
                ue_hbm.at[pl.ds(0, 8), :], ubuf.at[b, t], sem_u).wait()
            pltpu.make_async_copy(
                me_hbm.at[pl.ds(0, 8), :], mbuf.at[b, t], sem_m).wait()

    for b in range(NBUF):
        issue(b, b)

    def body(i, acc):
        for b in range(NBUF):
            g = i * NBUF + b
            iv_u = idx_u_v[pl.ds(g * L, L)]
            iv_m = idx_m_v[pl.ds(g * L, L)]
            wait_group(b)
            for t in range(L):
                ru = iv_u[t] & 7
                rm = iv_m[t] & 7
                for q in range(D // L):
                    acc = acc + (ubuf[b, t, ru, pl.ds(q * L, L)]
                                 * mbuf[b, t, rm, pl.ds(q * L, L)])

            @pl.when(g + NBUF < GROUPS)
            def _():
                issue(g + NBUF, b)
        return acc

    acc = lax.fori_loop(0, GROUPS // NBUF, body, jnp.zeros((L,), jnp.float32))
    stage_v[...] = acc
    pltpu.sync_copy(stage_v, part_hbm.at[ot])


def _out_body(uidx_hbm, midx_hbm, ub_hbm, mb_hbm, gb_hbm, part_hbm, out_hbm,
              idx_u_v, idx_m_v, ubf, mbf, out_v, part_v, gb_v,
              sem_u, sem_m):
    c = lax.axis_index("c")
    s = lax.axis_index("s")
    ot = c * NS + s

    pltpu.sync_copy(part_hbm, part_v)
    pltpu.sync_copy(gb_hbm, gb_v)
    for j in range(ROWS_PER_TILE // 128):
        pltpu.sync_copy(uidx_hbm.at[ot * 4 + j], idx_u_v.at[pl.ds(j * 128, 128)])
        pltpu.sync_copy(midx_hbm.at[ot * 4 + j], idx_m_v.at[pl.ds(j * 128, 128)])
    descs = []
    for j in range(ROWS_PER_TILE // 128):
        sl = pl.ds(j * 128, 128)
        descs.append(pltpu.async_copy(
            ub_hbm.at[idx_u_v.at[sl]], ubf.at[sl], sem_u))
        descs.append(pltpu.async_copy(
            mb_hbm.at[idx_m_v.at[sl]], mbf.at[sl], sem_m))

    tot = part_v[0]
    for q in range(1, NC * NS):
        tot = tot + part_v[q]
    dot = tot[0]
    for q in range(1, L):
        dot = dot + tot[q]
    base = dot + gb_v[...]

    for d in descs:
        d.wait()
    for k in range(ROWS_PER_TILE // L):
        sl = pl.ds(k * L, L)
        out_v[sl] = ubf[sl] + mbf[sl] + base
    pltpu.sync_copy(out_v, out_hbm.at[pl.ds(ot * ROWS_PER_TILE, ROWS_PER_TILE)])


@jax.jit
def _run(uidx_r, midx_r, user_emb, movie_emb, ub2d, mb2d, gb_vec):
    mesh = plsc.VectorSubcoreMesh(core_axis_name="c", subcore_axis_name="s")
    part = functools.partial(
        pl.kernel,
        out_type=jax.ShapeDtypeStruct((NC * NS, L), jnp.float32),
        mesh=mesh,
        compiler_params=pltpu.CompilerParams(use_tc_tiling_on_sc=True),
        scratch_types=[
            pltpu.VMEM((ROWS_PER_TILE,), jnp.int32),
            pltpu.VMEM((ROWS_PER_TILE,), jnp.int32),
            pltpu.VMEM((NBUF, L, 8, D), jnp.float32),
            pltpu.VMEM((NBUF, L, 8, D), jnp.float32),
            pltpu.VMEM((L,), jnp.float32),
            pltpu.SemaphoreType.DMA,
            pltpu.SemaphoreType.DMA,
        ],
    )(_dot_body)(uidx_r, midx_r, user_emb, movie_emb)

    out = functools.partial(
        pl.kernel,
        out_type=jax.ShapeDtypeStruct((B,), jnp.float32),
        mesh=mesh,
        compiler_params=pltpu.CompilerParams(use_tc_tiling_on_sc=False),
        scratch_types=[
            pltpu.VMEM((ROWS_PER_TILE,), jnp.int32),
            pltpu.VMEM((ROWS_PER_TILE,), jnp.int32),
            pltpu.VMEM((ROWS_PER_TILE,), jnp.float32),
            pltpu.VMEM((ROWS_PER_TILE,), jnp.float32),
            pltpu.VMEM((ROWS_PER_TILE,), jnp.float32),
            pltpu.VMEM((NC * NS, L), jnp.float32),
            pltpu.VMEM((L,), jnp.float32),
            pltpu.SemaphoreType.DMA,
            pltpu.SemaphoreType.DMA,
        ],
    )(_out_body)(uidx_r, midx_r, ub2d, mb2d, gb_vec, part)
    return out


def kernel(inputs, user_emb, movie_emb, user_bias_table, movie_bias_table,
           global_bias):
    uidx = inputs[:, 0].reshape(B // 128, 128)
    midx = inputs[:, 1].reshape(B // 128, 128)
    gb_vec = jnp.full((L,), global_bias, dtype=jnp.float32)
    return _run(uidx, midx, user_emb, movie_emb,
                user_bias_table.reshape(-1), movie_bias_table.reshape(-1),
                gb_vec)
